# Initial kernel scaffold; baseline (speedup 1.0000x reference)
import jax
import jax.numpy as jnp
from jax.experimental import pallas as pl


def kernel(x, weights):
    raise NotImplementedError("write your pallas kernel here")


# speedup vs baseline: 8.9248x; 8.9248x over previous
import jax
import jax.numpy as jnp
import numpy as np
from jax.experimental import pallas as pl

_N_CONN = 128
_OUT_SIZE = 1024
_BM = 1024

_S_NP = np.zeros((_N_CONN, _OUT_SIZE), dtype=np.float32)
for _i in range(_N_CONN):
    _S_NP[_i, _i % 64] = 1.0


def _dense_kernel(x_ref, w_ref, s_ref, o_ref):
    o_ref[...] = jnp.dot(
        x_ref[...] * w_ref[...],
        s_ref[...],
        preferred_element_type=jnp.float32,
    )


def kernel(x, weights):
    b = x.shape[0]
    w_row = weights.reshape(1, _N_CONN)
    s = jnp.asarray(_S_NP)
    return pl.pallas_call(
        _dense_kernel,
        grid=(b // _BM,),
        in_specs=[
            pl.BlockSpec((_BM, _N_CONN), lambda i: (i, 0)),
            pl.BlockSpec((1, _N_CONN), lambda i: (0, 0)),
            pl.BlockSpec((_N_CONN, _OUT_SIZE), lambda i: (0, 0)),
        ],
        out_specs=pl.BlockSpec((_BM, _OUT_SIZE), lambda i: (i, 0)),
        out_shape=jax.ShapeDtypeStruct((b, _OUT_SIZE), jnp.float32),
    )(x, w_row, s)
